# baseline (speedup 1.0000x reference)
import functools
import math

import jax
import jax.numpy as jnp
from jax import lax
from jax.experimental import pallas as pl
from jax.experimental.pallas import tpu as pltpu
from jax.experimental.pallas import tpu_sc as plsc

_SCALING = 30.0
_MARGIN = 0.5
_COS_M = math.cos(_MARGIN)
_SIN_M = math.sin(_MARGIN)
_THRESH = -math.cos(_MARGIN)
_MMV = math.sin(_MARGIN) * _MARGIN
_C1 = _SCALING / math.log(2.0)

_CB = 2048
_SC_LANES = 16
_ROW_W = 128


def _sc_gather_kernel(table_ref, label_ref, out_ref, lbl_v, idx_v, rows_v,
                      sem, *, bpw, ncols, num_cores):
    wid = lax.axis_index("s") * num_cores + lax.axis_index("c")
    base = wid * bpw
    pltpu.sync_copy(label_ref.at[pl.ds(base, bpw)], lbl_v)
    iota = lax.iota(jnp.int32, _SC_LANES)
    for k in range(bpw // _SC_LANES):
        lbl = lbl_v[pl.ds(k * _SC_LANES, _SC_LANES)]
        flat = (base + k * _SC_LANES + iota) * ncols + lbl
        idx_v[pl.ds(k * _SC_LANES, _SC_LANES)] = lax.shift_right_logical(flat, 7)
    pltpu.async_copy(table_ref.at[idx_v], rows_v, sem).wait()
    pltpu.sync_copy(rows_v, out_ref.at[pl.ds(base, bpw)])


def _reduce_kernel(cosine_ref, sum_ref, acc_ref, *, nfull, ncols):
    step = pl.program_id(0)
    nsteps = pl.num_programs(0)

    @pl.when(step == 0)
    def _init():
        acc_ref[...] = jnp.zeros_like(acc_ref)

    @pl.when(step < nfull)
    def _full():
        x = cosine_ref[...]
        acc_ref[...] += jnp.sum(jnp.exp2(x * _C1 - _C1), axis=1, keepdims=True)

    @pl.when(step >= nfull)
    def _tail():
        x = cosine_ref[...]
        col = step * _CB + lax.broadcasted_iota(jnp.int32, x.shape, 1)
        e = jnp.where(col < ncols, jnp.exp2(x * _C1 - _C1), 0.0)
        acc_ref[...] += jnp.sum(e, axis=1, keepdims=True)

    @pl.when(step == nsteps - 1)
    def _fin():
        sum_ref[...] = acc_ref[...]


def _combine_kernel(sum_ref, rows_ref, label_ref, out_ref, *, ncols):
    s = sum_ref[...]
    rows = rows_ref[...]
    ri = lax.broadcasted_iota(jnp.int32, label_ref.shape, 0)
    lane = jnp.bitwise_and(ri * ncols + label_ref[...], _ROW_W - 1)
    li = lax.broadcasted_iota(jnp.int32, rows.shape, 1)
    t = jnp.sum(jnp.where(li == lane, rows, 0.0), axis=1, keepdims=True)
    tc = jnp.clip(t, -1.0, 1.0)
    tr = jnp.where(
        t > _THRESH,
        tc * _COS_M - jnp.sqrt(jnp.maximum(1.0 - tc * tc, 0.0)) * _SIN_M,
        t - _MMV,
    )
    s2 = s - jnp.exp2(t * _C1 - _C1) + jnp.exp2(tr * _C1 - _C1)
    ce = jnp.log(s2) - (tr * _SCALING - _SCALING)
    p = jnp.exp(-ce)
    loss = (1.0 - p) * ce
    out_ref[...] = jnp.sum(loss, keepdims=True) / loss.shape[0]


def _gather_targets(cosine, label):
    b, c = cosine.shape
    info = plsc.get_sparse_core_info()
    num_workers = info.num_cores * info.num_subcores
    bpw = b // num_workers
    table = cosine.reshape(b * c // _ROW_W, _ROW_W)
    mesh = plsc.VectorSubcoreMesh(core_axis_name="c", subcore_axis_name="s")
    grab = functools.partial(
        pl.kernel,
        mesh=mesh,
        out_type=jax.ShapeDtypeStruct((b, _ROW_W), jnp.float32),
        scratch_types=[
            pltpu.VMEM((bpw,), jnp.int32),
            pltpu.VMEM((bpw,), jnp.int32),
            pltpu.VMEM((bpw, _ROW_W), jnp.float32),
            pltpu.SemaphoreType.DMA,
        ],
    )(functools.partial(
        _sc_gather_kernel,
        bpw=bpw,
        ncols=c,
        num_cores=info.num_cores,
    ))
    return grab(table, label)


def kernel(cosine, label):
    b, c = cosine.shape
    label = label.astype(jnp.int32)
    trows = _gather_targets(cosine, label)

    nfull = c // _CB
    nsteps = nfull + (1 if c % _CB else 0)
    row_sums = pl.pallas_call(
        functools.partial(_reduce_kernel, nfull=nfull, ncols=c),
        grid=(nsteps,),
        in_specs=[pl.BlockSpec((b, _CB), lambda i: (0, i))],
        out_specs=pl.BlockSpec((b, 1), lambda i: (0, 0)),
        out_shape=jax.ShapeDtypeStruct((b, 1), jnp.float32),
        scratch_shapes=[pltpu.VMEM((b, 1), jnp.float32)],
    )(cosine)

    out = pl.pallas_call(
        functools.partial(_combine_kernel, ncols=c),
        in_specs=[
            pl.BlockSpec((b, 1), lambda: (0, 0)),
            pl.BlockSpec((b, _ROW_W), lambda: (0, 0)),
            pl.BlockSpec((b, 1), lambda: (0, 0)),
        ],
        out_specs=pl.BlockSpec((1, 1), lambda: (0, 0)),
        out_shape=jax.ShapeDtypeStruct((1, 1), jnp.float32),
    )(row_sums, trows, label.reshape(b, 1))
    return out[0, 0]


# speedup vs baseline: 2.1532x; 2.1532x over previous
import functools
import math

import jax
import jax.numpy as jnp
from jax import lax
from jax.experimental import pallas as pl
from jax.experimental.pallas import tpu as pltpu
from jax.experimental.pallas import tpu_sc as plsc

_SCALING = 30.0
_MARGIN = 0.5
_COS_M = math.cos(_MARGIN)
_SIN_M = math.sin(_MARGIN)
_THRESH = -math.cos(_MARGIN)
_MMV = math.sin(_MARGIN) * _MARGIN
_C1 = _SCALING / math.log(2.0)

_CB = 2048
_SC_LANES = 16
_ROW_W = 128
_SUBL = 8


def _sc_gather_kernel(cos_ref, label_ref, out_ref, lbl_v, tiles_v,
                      rows_v, sem, *, bpw, num_cores):
    wid = lax.axis_index("s") * num_cores + lax.axis_index("c")
    base = wid * bpw
    pltpu.sync_copy(label_ref.at[pl.ds(base, bpw)], lbl_v)
    copies = []
    for j in range(bpw):
        lvec = lbl_v[pl.ds((j // _SC_LANES) * _SC_LANES, _SC_LANES)]
        col0 = pl.multiple_of(
            lax.bitwise_and(lvec[j % _SC_LANES], -_ROW_W), _ROW_W)
        row0 = base + (j // _SUBL) * _SUBL
        copies.append(pltpu.async_copy(
            cos_ref.at[pl.ds(row0, _SUBL), pl.ds(col0, _ROW_W)],
            tiles_v.at[j], sem))
    for cp in copies:
        cp.wait()
    for j in range(bpw):
        for kk in range(_ROW_W // _SC_LANES):
            rows_v[j, pl.ds(kk * _SC_LANES, _SC_LANES)] = (
                tiles_v[j, j % _SUBL, pl.ds(kk * _SC_LANES, _SC_LANES)])
    pltpu.sync_copy(rows_v, out_ref.at[pl.ds(base, bpw)])


def _reduce_kernel(cosine_ref, sum_ref, acc_ref, *, nfull, ncols):
    step = pl.program_id(0)
    nsteps = pl.num_programs(0)

    @pl.when(step == 0)
    def _init():
        acc_ref[...] = jnp.zeros_like(acc_ref)

    @pl.when(step < nfull)
    def _full():
        x = cosine_ref[...]
        acc_ref[...] += jnp.sum(jnp.exp2(x * _C1 - _C1), axis=1, keepdims=True)

    @pl.when(step >= nfull)
    def _tail():
        x = cosine_ref[...]
        col = step * _CB + lax.broadcasted_iota(jnp.int32, x.shape, 1)
        e = jnp.where(col < ncols, jnp.exp2(x * _C1 - _C1), 0.0)
        acc_ref[...] += jnp.sum(e, axis=1, keepdims=True)

    @pl.when(step == nsteps - 1)
    def _fin():
        sum_ref[...] = acc_ref[...]


def _combine_kernel(sum_ref, rows_ref, label_ref, out_ref, *, ncols):
    s = sum_ref[...]
    rows = rows_ref[...]
    lane = jnp.bitwise_and(label_ref[...], _ROW_W - 1)
    li = lax.broadcasted_iota(jnp.int32, rows.shape, 1)
    t = jnp.sum(jnp.where(li == lane, rows, 0.0), axis=1, keepdims=True)
    tc = jnp.clip(t, -1.0, 1.0)
    tr = jnp.where(
        t > _THRESH,
        tc * _COS_M - jnp.sqrt(jnp.maximum(1.0 - tc * tc, 0.0)) * _SIN_M,
        t - _MMV,
    )
    s2 = s - jnp.exp2(t * _C1 - _C1) + jnp.exp2(tr * _C1 - _C1)
    ce = jnp.log(s2) - (tr * _SCALING - _SCALING)
    p = jnp.exp(-ce)
    loss = (1.0 - p) * ce
    out_ref[...] = jnp.sum(loss, keepdims=True) / loss.shape[0]


def _gather_targets(cosine, label):
    b, c = cosine.shape
    info = plsc.get_sparse_core_info()
    num_workers = info.num_cores * info.num_subcores
    bpw = b // num_workers
    mesh = plsc.VectorSubcoreMesh(core_axis_name="c", subcore_axis_name="s")
    grab = functools.partial(
        pl.kernel,
        mesh=mesh,
        out_type=jax.ShapeDtypeStruct((b, _ROW_W), jnp.float32),
        scratch_types=[
            pltpu.VMEM((bpw,), jnp.int32),
            pltpu.VMEM((bpw, _SUBL, _ROW_W), jnp.float32),
            pltpu.VMEM((bpw, _ROW_W), jnp.float32),
            pltpu.SemaphoreType.DMA,
        ],
    )(functools.partial(
        _sc_gather_kernel,
        bpw=bpw,
        num_cores=info.num_cores,
    ))
    return grab(cosine, label)


def kernel(cosine, label):
    b, c = cosine.shape
    label = label.astype(jnp.int32)
    trows = _gather_targets(cosine, label)

    nfull = c // _CB
    nsteps = nfull + (1 if c % _CB else 0)
    row_sums = pl.pallas_call(
        functools.partial(_reduce_kernel, nfull=nfull, ncols=c),
        grid=(nsteps,),
        in_specs=[pl.BlockSpec((b, _CB), lambda i: (0, i))],
        out_specs=pl.BlockSpec((b, 1), lambda i: (0, 0)),
        out_shape=jax.ShapeDtypeStruct((b, 1), jnp.float32),
        scratch_shapes=[pltpu.VMEM((b, 1), jnp.float32)],
    )(cosine)

    out = pl.pallas_call(
        functools.partial(_combine_kernel, ncols=c),
        in_specs=[
            pl.BlockSpec((b, 1), lambda: (0, 0)),
            pl.BlockSpec((b, _ROW_W), lambda: (0, 0)),
            pl.BlockSpec((b, 1), lambda: (0, 0)),
        ],
        out_specs=pl.BlockSpec((1, 1), lambda: (0, 0)),
        out_shape=jax.ShapeDtypeStruct((1, 1), jnp.float32),
    )(row_sums, trows, label.reshape(b, 1))
    return out[0, 0]
